# Initial kernel scaffold; baseline (speedup 1.0000x reference)
import jax
import jax.numpy as jnp
from jax.experimental import pallas as pl


def kernel(features, W_sal, b_sal, A, Wm, bm):
    raise NotImplementedError("write your pallas kernel here")


# speedup vs baseline: 5.4159x; 5.4159x over previous
import functools

import jax
import jax.numpy as jnp
from jax import lax
from jax.experimental import pallas as pl
from jax.experimental.pallas import tpu as pltpu

_P = 4
_K = 2 * _P


def _k1_body(feat_ref, wsal_ref, bsal_ref, sal_ref, pos_ref, mask_ref,
             gath_ref, *, hw, w, t_steps):
    feat = feat_ref[0]
    wsal = wsal_ref[...]
    b = bsal_ref[0]
    sal = jnp.dot(feat, wsal, preferred_element_type=jnp.float32) + b
    sal_ref[0] = sal

    iota = lax.broadcasted_iota(jnp.int32, (hw, 1), 0)
    big = jnp.int32(1 << 30)
    idxs = []
    for _ in range(_K):
        m = jnp.max(sal)
        idx = jnp.min(jnp.where(sal == m, iota, big))
        idxs.append(idx)
        sal = jnp.where(iota == idx, -jnp.inf, sal)

    r_all = [i // w for i in idxs]
    c_all = [i % w for i in idxs]

    for p in range(_P):
        r0, c0 = r_all[p], c_all[p]
        r1, c1 = r_all[p + _P], c_all[p + _P]
        dr = jnp.abs(r1 - r0)
        dc = jnp.abs(c1 - c0)
        sr = jnp.where(r1 >= r0, 1, -1).astype(jnp.int32)
        sc = jnp.where(c1 >= c0, 1, -1).astype(jnp.int32)

        def step(t, carry, r1=r1, c1=c1, dr=dr, dc=dc, sr=sr, sc=sc, p=p):
            r, c, err, active = carry
            pos_ref[0, 0, t * _P + p] = r * w + c
            mask_ref[0, 0, t * _P + p] = jnp.where(active, 1.0, 0.0)
            at_end = (r == r1) & (c == c1)
            nxt = active & (~at_end)
            e2 = 2 * err
            cond1 = e2 > -dc
            cond2 = e2 < dr
            err_n = err - jnp.where(cond1, dc, 0) + jnp.where(cond2, dr, 0)
            r_n = r + jnp.where(cond1, sr, 0)
            c_n = c + jnp.where(cond2, sc, 0)
            r2 = jnp.where(nxt, r_n, r)
            c2 = jnp.where(nxt, c_n, c)
            err2 = jnp.where(nxt, err_n, err)
            return (r2, c2, err2, nxt)

        lax.fori_loop(0, t_steps, step,
                      (r0, c0, dr - dc, jnp.bool_(True)))

    def gath(row, _):
        pos = pos_ref[0, 0, row]
        mk = mask_ref[0, 0, row]
        gath_ref[0, pl.ds(row, 1), :] = feat_ref[0, pl.ds(pos, 1), :] * mk
        return 0
    lax.fori_loop(0, t_steps * _P, gath, 0)


def _k2_body(a_ref, wm_ref, bm_ref, x_ref, y_ref, *, t_steps, bp):
    a = 1.0 / (1.0 + jnp.exp(-a_ref[...]))
    bm = bm_ref[...]
    wm = wm_ref[...]
    h0 = jnp.zeros((bp, wm.shape[0]), jnp.float32)

    def step(t, h):
        x = x_ref[t]
        h2 = a * h + jnp.dot(x, wm, preferred_element_type=jnp.float32) + bm
        y_ref[t] = h2
        return h2

    lax.fori_loop(0, t_steps, step, h0)


def _k3_body(m_ref, pos_ref, mask_ref, corr_ref, counts, *, hw, t_steps):
    counts[...] = jnp.zeros(counts.shape, jnp.float32)
    corr_ref[0] = jnp.zeros(corr_ref.shape[1:], jnp.float32)

    def scat(row, _):
        pos = pos_ref[0, 0, row]
        mk = mask_ref[0, 0, row]
        corr_ref[0, pl.ds(pos, 1), :] = (
            corr_ref[0, pl.ds(pos, 1), :] + m_ref[0, pl.ds(row, 1), :] * mk)
        counts[pl.ds(pos, 1), :] = counts[pl.ds(pos, 1), :] + mk
        return 0
    lax.fori_loop(0, t_steps * _P, scat, 0)

    corr_ref[0] = corr_ref[0] / jnp.maximum(counts[...], 1.0)


@jax.jit
def kernel(features, W_sal, b_sal, A, Wm, bm):
    B, C, H, W = features.shape
    HW = H * W
    T = max(H, W)
    PT = _P * T

    feat_hwc = features.reshape(B, C, HW).transpose(0, 2, 1)

    sal, pos, mask, gathered = pl.pallas_call(
        functools.partial(_k1_body, hw=HW, w=W, t_steps=T),
        grid=(B,),
        in_specs=[
            pl.BlockSpec((1, HW, C), lambda b: (b, 0, 0)),
            pl.BlockSpec((C, 1), lambda b: (0, 0)),
            pl.BlockSpec(memory_space=pltpu.SMEM),
        ],
        out_specs=[
            pl.BlockSpec((1, HW, 1), lambda b: (b, 0, 0)),
            pl.BlockSpec((1, 1, PT), lambda b: (b, 0, 0), memory_space=pltpu.SMEM),
            pl.BlockSpec((1, 1, PT), lambda b: (b, 0, 0), memory_space=pltpu.SMEM),
            pl.BlockSpec((1, PT, C), lambda b: (b, 0, 0)),
        ],
        out_shape=[
            jax.ShapeDtypeStruct((B, HW, 1), jnp.float32),
            jax.ShapeDtypeStruct((B, 1, PT), jnp.int32),
            jax.ShapeDtypeStruct((B, 1, PT), jnp.float32),
            jax.ShapeDtypeStruct((B, PT, C), jnp.float32),
        ],
    )(feat_hwc, W_sal.reshape(C, 1), b_sal.reshape(1))

    x_tm = gathered.reshape(B, T, _P, C).transpose(1, 0, 2, 3).reshape(T, B * _P, C)

    ys = pl.pallas_call(
        functools.partial(_k2_body, t_steps=T, bp=B * _P),
        in_specs=[
            pl.BlockSpec((1, C), lambda: (0, 0)),
            pl.BlockSpec((C, C), lambda: (0, 0)),
            pl.BlockSpec((1, C), lambda: (0, 0)),
            pl.BlockSpec((T, B * _P, C), lambda: (0, 0, 0)),
        ],
        out_specs=pl.BlockSpec((T, B * _P, C), lambda: (0, 0, 0)),
        out_shape=jax.ShapeDtypeStruct((T, B * _P, C), jnp.float32),
    )(A.reshape(1, C), Wm, bm.reshape(1, C), x_tm)

    mout = ys.reshape(T, B, _P, C).transpose(1, 0, 2, 3).reshape(B, PT, C)

    corr_hwc = pl.pallas_call(
        functools.partial(_k3_body, hw=HW, t_steps=T),
        grid=(B,),
        in_specs=[
            pl.BlockSpec((1, PT, C), lambda b: (b, 0, 0)),
            pl.BlockSpec((1, 1, PT), lambda b: (b, 0, 0), memory_space=pltpu.SMEM),
            pl.BlockSpec((1, 1, PT), lambda b: (b, 0, 0), memory_space=pltpu.SMEM),
        ],
        out_specs=pl.BlockSpec((1, HW, C), lambda b: (b, 0, 0)),
        out_shape=jax.ShapeDtypeStruct((B, HW, C), jnp.float32),
        scratch_shapes=[pltpu.VMEM((HW, 1), jnp.float32)],
    )(mout, pos, mask)

    corrections = corr_hwc.transpose(0, 2, 1).reshape(B, C, H, W)
    sal_maps = sal.reshape(B, H, W)
    avg_path_len = jnp.sum(mask) / B
    return (corrections, avg_path_len, sal_maps)


# speedup vs baseline: 5.5259x; 1.0203x over previous
import functools

import jax
import jax.numpy as jnp
from jax import lax
from jax.experimental import pallas as pl
from jax.experimental.pallas import tpu as pltpu

_P = 4
_K = 2 * _P


def _body(feat_ref, wsal_ref, bsal_ref, a_ref, wm_ref, bm_ref,
          corr_ref, sal_ref, len_ref, posv, maskv, mout_ref,
          *, hw, w, t_steps):
    feat = feat_ref[0]
    wsal = wsal_ref[...]
    b = bsal_ref[0]

    sal = lax.dot_general(wsal, feat, (((1,), (0,)), ((), ())),
                          preferred_element_type=jnp.float32, precision=lax.Precision.HIGHEST) + b
    sal_ref[0] = sal

    iota = lax.broadcasted_iota(jnp.int32, (1, hw), 1)
    big = jnp.int32(1 << 30)
    idxs = []
    for _ in range(_K):
        m = jnp.max(sal)
        idx = jnp.min(jnp.where(sal == m, iota, big))
        idxs.append(idx)
        sal = jnp.where(iota == idx, -jnp.inf, sal)

    p_iota = lax.broadcasted_iota(jnp.int32, (_P, 1), 0)
    zeros4 = jnp.zeros((_P, 1), jnp.int32)

    def pack4(scalars):
        v = zeros4
        for p, s in enumerate(scalars):
            v = jnp.where(p_iota == p, s, v)
        return v

    r0 = pack4([i // w for i in idxs[:_P]])
    c0 = pack4([i % w for i in idxs[:_P]])
    r1 = pack4([i // w for i in idxs[_P:]])
    c1 = pack4([i % w for i in idxs[_P:]])

    dr = jnp.abs(r1 - r0)
    dc = jnp.abs(c1 - c0)
    sr = jnp.where(r1 >= r0, 1, -1).astype(jnp.int32)
    sc = jnp.where(c1 >= c0, 1, -1).astype(jnp.int32)

    r, c, err = r0, c0, dr - dc
    active = jnp.ones((_P, 1), jnp.bool_)
    for t in range(t_steps):
        posv[pl.ds(t * _P, _P), :] = r * w + c
        maskv[pl.ds(t * _P, _P), :] = jnp.where(active, 1.0, 0.0)
        at_end = (r == r1) & (c == c1)
        nxt = active & (~at_end)
        e2 = 2 * err
        cond1 = e2 > -dc
        cond2 = e2 < dr
        err_n = err - jnp.where(cond1, dc, 0) + jnp.where(cond2, dr, 0)
        r_n = r + jnp.where(cond1, sr, 0)
        c_n = c + jnp.where(cond2, sc, 0)
        r = jnp.where(nxt, r_n, r)
        c = jnp.where(nxt, c_n, c)
        err = jnp.where(nxt, err_n, err)
        active = nxt

    pos = posv[...]
    mk = maskv[...]
    len_ref[0, 0, 0] = jnp.sum(mk)

    col_iota = lax.broadcasted_iota(jnp.int32, (t_steps * _P, hw), 1)
    s_mat = jnp.where(col_iota == pos, 1.0, 0.0) * mk

    gathered = lax.dot_general(s_mat, feat, (((1,), (1,)), ((), ())),
                               preferred_element_type=jnp.float32, precision=lax.Precision.HIGHEST)

    u_all = lax.dot_general(gathered, wm_ref[...], (((1,), (0,)), ((), ())),
                            preferred_element_type=jnp.float32, precision=lax.Precision.HIGHEST) + bm_ref[...]

    a = 1.0 / (1.0 + jnp.exp(-a_ref[...]))
    h = jnp.zeros((_P, u_all.shape[1]), jnp.float32)
    for t in range(t_steps):
        h = a * h + u_all[t * _P:(t + 1) * _P, :]
        mout_ref[pl.ds(t * _P, _P), :] = h

    mout = mout_ref[...]
    corr = lax.dot_general(mout, s_mat, (((0,), (0,)), ((), ())),
                           preferred_element_type=jnp.float32, precision=lax.Precision.HIGHEST)
    counts = lax.dot_general(mk, s_mat, (((0,), (0,)), ((), ())),
                             preferred_element_type=jnp.float32, precision=lax.Precision.HIGHEST)
    corr_ref[0] = corr / jnp.maximum(counts, 1.0)


@jax.jit
def kernel(features, W_sal, b_sal, A, Wm, bm):
    B, C, H, W = features.shape
    HW = H * W
    T = max(H, W)
    PT = _P * T

    feat = features.reshape(B, C, HW)

    corr, sal, lens = pl.pallas_call(
        functools.partial(_body, hw=HW, w=W, t_steps=T),
        grid=(B,),
        in_specs=[
            pl.BlockSpec((1, C, HW), lambda b: (b, 0, 0)),
            pl.BlockSpec((1, C), lambda b: (0, 0)),
            pl.BlockSpec(memory_space=pltpu.SMEM),
            pl.BlockSpec((1, C), lambda b: (0, 0)),
            pl.BlockSpec((C, C), lambda b: (0, 0)),
            pl.BlockSpec((1, C), lambda b: (0, 0)),
        ],
        out_specs=[
            pl.BlockSpec((1, C, HW), lambda b: (b, 0, 0)),
            pl.BlockSpec((1, 1, HW), lambda b: (b, 0, 0)),
            pl.BlockSpec((1, 1, 1), lambda b: (b, 0, 0),
                         memory_space=pltpu.SMEM),
        ],
        out_shape=[
            jax.ShapeDtypeStruct((B, C, HW), jnp.float32),
            jax.ShapeDtypeStruct((B, 1, HW), jnp.float32),
            jax.ShapeDtypeStruct((B, 1, 1), jnp.float32),
        ],
        scratch_shapes=[
            pltpu.VMEM((PT, 1), jnp.int32),
            pltpu.VMEM((PT, 1), jnp.float32),
            pltpu.VMEM((PT, C), jnp.float32),
        ],
    )(feat, W_sal.reshape(1, C), b_sal.reshape(1), A.reshape(1, C),
      Wm, bm.reshape(1, C))

    corrections = corr.reshape(B, C, H, W)
    sal_maps = sal.reshape(B, H, W)
    avg_path_len = jnp.sum(lens) / B
    return (corrections, avg_path_len, sal_maps)


# speedup vs baseline: 5.7803x; 1.0460x over previous
import functools

import jax
import jax.numpy as jnp
from jax import lax
from jax.experimental import pallas as pl
from jax.experimental.pallas import tpu as pltpu

_P = 4
_K = 2 * _P


def _body(feat_ref, wsal_ref, bsal_ref, a_ref, wm_ref, bm_ref,
          corr_ref, sal_ref, len_ref, posv, maskv, mout_ref,
          *, hw, w, t_steps):
    feat = feat_ref[0]
    wsal = wsal_ref[...]
    b = bsal_ref[0]

    sal = lax.dot_general(wsal, feat, (((1,), (0,)), ((), ())),
                          preferred_element_type=jnp.float32) + b
    sal_ref[0] = sal

    iota = lax.broadcasted_iota(jnp.int32, (1, hw), 1)
    big = jnp.int32(1 << 30)
    idxs = []
    for _ in range(_K):
        m = jnp.max(sal)
        idx = jnp.min(jnp.where(sal == m, iota, big))
        idxs.append(idx)
        sal = jnp.where(iota == idx, -jnp.inf, sal)

    p_iota = lax.broadcasted_iota(jnp.int32, (_P, 1), 0)
    zeros4 = jnp.zeros((_P, 1), jnp.int32)

    def pack4(scalars):
        v = zeros4
        for p, s in enumerate(scalars):
            v = jnp.where(p_iota == p, s, v)
        return v

    r0 = pack4([i // w for i in idxs[:_P]])
    c0 = pack4([i % w for i in idxs[:_P]])
    r1 = pack4([i // w for i in idxs[_P:]])
    c1 = pack4([i % w for i in idxs[_P:]])

    dr = jnp.abs(r1 - r0)
    dc = jnp.abs(c1 - c0)
    sr = jnp.where(r1 >= r0, 1, -1).astype(jnp.int32)
    sc = jnp.where(c1 >= c0, 1, -1).astype(jnp.int32)

    r, c, err = r0, c0, dr - dc
    active = jnp.ones((_P, 1), jnp.bool_)
    for t in range(t_steps):
        posv[pl.ds(t * _P, _P), :] = r * w + c
        maskv[pl.ds(t * _P, _P), :] = jnp.where(active, 1.0, 0.0)
        at_end = (r == r1) & (c == c1)
        nxt = active & (~at_end)
        e2 = 2 * err
        cond1 = e2 > -dc
        cond2 = e2 < dr
        err_n = err - jnp.where(cond1, dc, 0) + jnp.where(cond2, dr, 0)
        r_n = r + jnp.where(cond1, sr, 0)
        c_n = c + jnp.where(cond2, sc, 0)
        r = jnp.where(nxt, r_n, r)
        c = jnp.where(nxt, c_n, c)
        err = jnp.where(nxt, err_n, err)
        active = nxt

    pos = posv[...]
    mk = maskv[...]
    len_ref[0, 0, 0] = jnp.sum(mk)

    col_iota = lax.broadcasted_iota(jnp.int32, (t_steps * _P, hw), 1)
    s_mat = jnp.where(col_iota == pos, 1.0, 0.0) * mk

    gathered = lax.dot_general(s_mat, feat, (((1,), (1,)), ((), ())),
                               preferred_element_type=jnp.float32, precision=lax.Precision.HIGHEST)

    u_all = lax.dot_general(gathered, wm_ref[...], (((1,), (0,)), ((), ())),
                            preferred_element_type=jnp.float32, precision=lax.Precision.HIGHEST) + bm_ref[...]

    a = 1.0 / (1.0 + jnp.exp(-a_ref[...]))
    h = jnp.zeros((_P, u_all.shape[1]), jnp.float32)
    for t in range(t_steps):
        h = a * h + u_all[t * _P:(t + 1) * _P, :]
        mout_ref[pl.ds(t * _P, _P), :] = h

    mout = mout_ref[...]
    corr = lax.dot_general(mout, s_mat, (((0,), (0,)), ((), ())),
                           preferred_element_type=jnp.float32, precision=lax.Precision.HIGHEST)
    counts = lax.dot_general(mk, s_mat, (((0,), (0,)), ((), ())),
                             preferred_element_type=jnp.float32, precision=lax.Precision.HIGHEST)
    corr_ref[0] = corr / jnp.maximum(counts, 1.0)


@jax.jit
def kernel(features, W_sal, b_sal, A, Wm, bm):
    B, C, H, W = features.shape
    HW = H * W
    T = max(H, W)
    PT = _P * T

    feat = features.reshape(B, C, HW)

    corr, sal, lens = pl.pallas_call(
        functools.partial(_body, hw=HW, w=W, t_steps=T),
        grid=(B,),
        in_specs=[
            pl.BlockSpec((1, C, HW), lambda b: (b, 0, 0)),
            pl.BlockSpec((1, C), lambda b: (0, 0)),
            pl.BlockSpec(memory_space=pltpu.SMEM),
            pl.BlockSpec((1, C), lambda b: (0, 0)),
            pl.BlockSpec((C, C), lambda b: (0, 0)),
            pl.BlockSpec((1, C), lambda b: (0, 0)),
        ],
        out_specs=[
            pl.BlockSpec((1, C, HW), lambda b: (b, 0, 0)),
            pl.BlockSpec((1, 1, HW), lambda b: (b, 0, 0)),
            pl.BlockSpec((1, 1, 1), lambda b: (b, 0, 0),
                         memory_space=pltpu.SMEM),
        ],
        out_shape=[
            jax.ShapeDtypeStruct((B, C, HW), jnp.float32),
            jax.ShapeDtypeStruct((B, 1, HW), jnp.float32),
            jax.ShapeDtypeStruct((B, 1, 1), jnp.float32),
        ],
        scratch_shapes=[
            pltpu.VMEM((PT, 1), jnp.int32),
            pltpu.VMEM((PT, 1), jnp.float32),
            pltpu.VMEM((PT, C), jnp.float32),
        ],
    )(feat, W_sal.reshape(1, C), b_sal.reshape(1), A.reshape(1, C),
      Wm, bm.reshape(1, C))

    corrections = corr.reshape(B, C, H, W)
    sal_maps = sal.reshape(B, H, W)
    avg_path_len = jnp.sum(lens) / B
    return (corrections, avg_path_len, sal_maps)


# speedup vs baseline: 7.9254x; 1.3711x over previous
import functools

import jax
import jax.numpy as jnp
from jax import lax
from jax.experimental import pallas as pl
from jax.experimental.pallas import tpu as pltpu

_P = 4
_K = 2 * _P


def _body(feat_ref, wsal_ref, bsal_ref, a_ref, wm_ref, bm_ref,
          corr_ref, sal_ref, len_ref, posv, maskv, mout_ref,
          *, hw, w, t_steps):
    feat = feat_ref[0]
    wsal = wsal_ref[...]
    b = bsal_ref[0]

    sal = lax.dot_general(wsal, feat, (((1,), (0,)), ((), ())),
                          preferred_element_type=jnp.float32) + b
    sal_ref[0] = sal

    iota = lax.broadcasted_iota(jnp.int32, (1, hw), 1)
    big = jnp.int32(1 << 30)
    idxs = []
    for _ in range(_K):
        m = jnp.max(sal)
        idx = jnp.min(jnp.where(sal == m, iota, big))
        idxs.append(idx)
        sal = jnp.where(iota == idx, -jnp.inf, sal)

    p_iota = lax.broadcasted_iota(jnp.int32, (_P, 1), 0)
    zeros4 = jnp.zeros((_P, 1), jnp.int32)

    def pack4(scalars):
        v = zeros4
        for p, s in enumerate(scalars):
            v = jnp.where(p_iota == p, s, v)
        return v

    r0 = pack4([i // w for i in idxs[:_P]])
    c0 = pack4([i % w for i in idxs[:_P]])
    r1 = pack4([i // w for i in idxs[_P:]])
    c1 = pack4([i % w for i in idxs[_P:]])

    dr = jnp.abs(r1 - r0)
    dc = jnp.abs(c1 - c0)
    sr = jnp.where(r1 >= r0, 1, -1).astype(jnp.int32)
    sc = jnp.where(c1 >= c0, 1, -1).astype(jnp.int32)

    r, c, err = r0, c0, dr - dc
    active = jnp.ones((_P, 1), jnp.bool_)
    for t in range(t_steps):
        posv[pl.ds(t * _P, _P), :] = r * w + c
        maskv[pl.ds(t * _P, _P), :] = jnp.where(active, 1.0, 0.0)
        at_end = (r == r1) & (c == c1)
        nxt = active & (~at_end)
        e2 = 2 * err
        cond1 = e2 > -dc
        cond2 = e2 < dr
        err_n = err - jnp.where(cond1, dc, 0) + jnp.where(cond2, dr, 0)
        r_n = r + jnp.where(cond1, sr, 0)
        c_n = c + jnp.where(cond2, sc, 0)
        r = jnp.where(nxt, r_n, r)
        c = jnp.where(nxt, c_n, c)
        err = jnp.where(nxt, err_n, err)
        active = nxt

    pos = posv[...]
    mk = maskv[...]
    len_ref[0, 0, 0] = jnp.sum(mk)

    col_iota = lax.broadcasted_iota(jnp.int32, (t_steps * _P, hw), 1)
    s_mat = jnp.where(col_iota == pos, 1.0, 0.0) * mk

    gathered = lax.dot_general(s_mat, feat, (((1,), (1,)), ((), ())),
                               preferred_element_type=jnp.float32)

    u_all = lax.dot_general(gathered, wm_ref[...], (((1,), (0,)), ((), ())),
                            preferred_element_type=jnp.float32) + bm_ref[...]

    a = 1.0 / (1.0 + jnp.exp(-a_ref[...]))
    h = jnp.zeros((_P, u_all.shape[1]), jnp.float32)
    for t in range(t_steps):
        h = a * h + u_all[t * _P:(t + 1) * _P, :]
        mout_ref[pl.ds(t * _P, _P), :] = h

    mout = mout_ref[...]
    corr = lax.dot_general(mout, s_mat, (((0,), (0,)), ((), ())),
                           preferred_element_type=jnp.float32)
    counts = lax.dot_general(mk, s_mat, (((0,), (0,)), ((), ())),
                             preferred_element_type=jnp.float32)
    corr_ref[0] = corr / jnp.maximum(counts, 1.0)


@jax.jit
def kernel(features, W_sal, b_sal, A, Wm, bm):
    B, C, H, W = features.shape
    HW = H * W
    T = max(H, W)
    PT = _P * T

    feat = features.reshape(B, C, HW)

    corr, sal, lens = pl.pallas_call(
        functools.partial(_body, hw=HW, w=W, t_steps=T),
        grid=(B,),
        in_specs=[
            pl.BlockSpec((1, C, HW), lambda b: (b, 0, 0)),
            pl.BlockSpec((1, C), lambda b: (0, 0)),
            pl.BlockSpec(memory_space=pltpu.SMEM),
            pl.BlockSpec((1, C), lambda b: (0, 0)),
            pl.BlockSpec((C, C), lambda b: (0, 0)),
            pl.BlockSpec((1, C), lambda b: (0, 0)),
        ],
        out_specs=[
            pl.BlockSpec((1, C, HW), lambda b: (b, 0, 0)),
            pl.BlockSpec((1, 1, HW), lambda b: (b, 0, 0)),
            pl.BlockSpec((1, 1, 1), lambda b: (b, 0, 0),
                         memory_space=pltpu.SMEM),
        ],
        out_shape=[
            jax.ShapeDtypeStruct((B, C, HW), jnp.float32),
            jax.ShapeDtypeStruct((B, 1, HW), jnp.float32),
            jax.ShapeDtypeStruct((B, 1, 1), jnp.float32),
        ],
        scratch_shapes=[
            pltpu.VMEM((PT, 1), jnp.int32),
            pltpu.VMEM((PT, 1), jnp.float32),
            pltpu.VMEM((PT, C), jnp.float32),
        ],
    )(feat, W_sal.reshape(1, C), b_sal.reshape(1), A.reshape(1, C),
      Wm, bm.reshape(1, C))

    corrections = corr.reshape(B, C, H, W)
    sal_maps = sal.reshape(B, H, W)
    avg_path_len = jnp.sum(lens) / B
    return (corrections, avg_path_len, sal_maps)
